# baseline (speedup 1.0000x reference)
import functools

import jax
import jax.numpy as jnp
import numpy as np
from jax.experimental import pallas as pl
from jax.experimental.pallas import tpu as pltpu

C = 1024
M = 64
K = 512
H = 8
B = 8
T = 8192
DH = C // H
TT = 512
NT = T // TT
SCALE = float(1.0 / np.sqrt(DH))


def _fused_kernel(bidx_ref, feat_ref, q_ref, ipw_ref, b3_ref,
                  ow_ref, ob_ref, lw_ref, lb_ref, out_ref,
                  qs_ref, numer_ref, denom_ref, cnt_ref, ctx_ref):
    i = pl.program_id(0)

    @pl.when(i == 0)
    def _init():
        q = jax.lax.dot_general(q_ref[...], ipw_ref[0:C, :],
                                (((1,), (1,)), ((), ())),
                                preferred_element_type=jnp.float32) + b3_ref[0:1, :]
        qs_ref[...] = q * SCALE
        numer_ref[...] = jnp.zeros_like(numer_ref)
        denom_ref[...] = jnp.zeros_like(denom_ref)
        cnt_ref[...] = jnp.zeros_like(cnt_ref)

    feat = feat_ref[...]
    k_t = jax.lax.dot_general(feat, ipw_ref[C:2 * C, :],
                              (((1,), (1,)), ((), ())),
                              preferred_element_type=jnp.float32) + b3_ref[1:2, :]
    v_t = jax.lax.dot_general(feat, ipw_ref[2 * C:3 * C, :],
                              (((1,), (1,)), ((), ())),
                              preferred_element_type=jnp.float32) + b3_ref[2:3, :]

    bidx = bidx_ref[0]
    lane_b = jax.lax.broadcasted_iota(jnp.int32, (TT, B * DH), 1) // DH
    ohx = (bidx == lane_b).astype(jnp.float32)
    oh = (bidx == jax.lax.broadcasted_iota(jnp.int32, (TT, B), 1)
          ).astype(jnp.float32)
    cnt_new = cnt_ref[...] + jnp.sum(oh, axis=0, keepdims=True)

    qs = qs_ref[...]
    nparts = []
    dparts = []
    for h in range(H):
        k_h = k_t[:, h * DH:(h + 1) * DH]
        v_h = v_t[:, h * DH:(h + 1) * DH]
        s_h = jax.lax.dot_general(qs[:, h * DH:(h + 1) * DH], k_h,
                                  (((1,), (1,)), ((), ())),
                                  preferred_element_type=jnp.float32)
        e_h = jnp.exp(s_h)
        vmask = jnp.concatenate([v_h] * B, axis=1) * ohx
        nparts.append(jnp.dot(e_h, vmask,
                              preferred_element_type=jnp.float32))
        dparts.append(jnp.dot(e_h, oh,
                              preferred_element_type=jnp.float32))
    numer_ref[...] += jnp.concatenate(nparts, axis=0)
    denom_ref[...] += jnp.concatenate(dparts, axis=0)
    cnt_ref[...] = cnt_new

    @pl.when(i == NT - 1)
    def _finalize():
        expmat = (jax.lax.broadcasted_iota(jnp.int32, (B, B * DH), 0)
                  == jax.lax.broadcasted_iota(jnp.int32, (B, B * DH), 1) // DH
                  ).astype(jnp.float32)
        d = denom_ref[...]
        inv = 1.0 / jnp.where(d == 0.0, 1.0, d)
        invx = jnp.dot(inv, expmat,
                       preferred_element_type=jnp.float32)
        ctxx = numer_ref[...] * invx
        for h in range(H):
            for b in range(B):
                ctx_ref[b * M:(b + 1) * M, h * DH:(h + 1) * DH] = (
                    ctxx[h * M:(h + 1) * M, b * DH:(b + 1) * DH])
        attn = jax.lax.dot_general(ctx_ref[...], ow_ref[...],
                                   (((1,), (1,)), ((), ())),
                                   preferred_element_type=jnp.float32) + ob_ref[...]
        outr = jax.lax.dot_general(attn, lw_ref[...], (((1,), (1,)), ((), ())),
                                   preferred_element_type=jnp.float32) + lb_ref[...]
        attn_d = jax.lax.dot_general(b3_ref[2:3, :], ow_ref[...],
                                     (((1,), (1,)), ((), ())),
                                     preferred_element_type=jnp.float32) + ob_ref[...]
        out_d = jax.lax.dot_general(attn_d, lw_ref[...],
                                    (((1,), (1,)), ((), ())),
                                    preferred_element_type=jnp.float32) + lb_ref[...]
        alive = (cnt_ref[...] > 0.0).astype(jnp.float32)
        rowsel = (jax.lax.broadcasted_iota(jnp.int32, (B * M, B), 0) // M
                  == jax.lax.broadcasted_iota(jnp.int32, (B * M, B), 1)
                  ).astype(jnp.float32)
        keep = jnp.sum(rowsel * alive, axis=1, keepdims=True)
        out_ref[...] = keep * outr + (1.0 - keep) * out_d


def _run(bidx3, feat, q2, ipw, b3, ow, ob2, lw, lb2):
    return pl.pallas_call(
        _fused_kernel,
        grid=(NT,),
        in_specs=[
            pl.BlockSpec((1, TT, 1), lambda i: (i, 0, 0)),
            pl.BlockSpec((TT, C), lambda i: (i, 0)),
            pl.BlockSpec((M, C), lambda i: (0, 0)),
            pl.BlockSpec((3 * C, C), lambda i: (0, 0)),
            pl.BlockSpec((3, C), lambda i: (0, 0)),
            pl.BlockSpec((C, C), lambda i: (0, 0)),
            pl.BlockSpec((1, C), lambda i: (0, 0)),
            pl.BlockSpec((K, C), lambda i: (0, 0)),
            pl.BlockSpec((1, K), lambda i: (0, 0)),
        ],
        out_specs=pl.BlockSpec((B * M, K), lambda i: (0, 0)),
        out_shape=jax.ShapeDtypeStruct((B * M, K), jnp.float32),
        scratch_shapes=[
            pltpu.VMEM((M, C), jnp.float32),
            pltpu.VMEM((H * M, B * DH), jnp.float32),
            pltpu.VMEM((H * M, B), jnp.float32),
            pltpu.VMEM((1, B), jnp.float32),
            pltpu.VMEM((B * M, C), jnp.float32),
        ],
        compiler_params=pltpu.CompilerParams(
            dimension_semantics=("arbitrary",),
        ),
    )(bidx3, feat, q2, ipw, b3, ow, ob2, lw, lb2)


def kernel(feat_all, batch_idx, queries, in_proj_w, in_proj_b, out_w, out_b,
           lin_w, lin_b):
    bidx3 = batch_idx.astype(jnp.int32).reshape(NT, TT, 1)
    q2 = queries.reshape(M, C)
    b3 = in_proj_b.reshape(3, C)
    ob2 = out_b.reshape(1, C)
    lb2 = lin_b.reshape(1, K)
    out = _run(bidx3, feat_all, q2, in_proj_w, b3, out_w, ob2, lin_w, lb2)
    return out.reshape(B, M, K)


# speedup vs baseline: 1.2559x; 1.2559x over previous
import functools

import jax
import jax.numpy as jnp
import numpy as np
from jax.experimental import pallas as pl
from jax.experimental.pallas import tpu as pltpu

C = 1024
M = 64
K = 512
H = 8
B = 8
T = 8192
DH = C // H
TT = 1024
NT = T // TT
SCALE = float(1.0 / np.sqrt(DH))


def _fused_kernel(bidx_ref, feat_ref, q_ref, ipw_ref, b3_ref,
                  owT_ref, ob_ref, lwT_ref, lb_ref, out_ref,
                  qs_ref, numer_ref, denom_ref, cnt_ref):
    i = pl.program_id(0)

    @pl.when(i == 0)
    def _init():
        q = jax.lax.dot_general(q_ref[...], ipw_ref[0:C, :],
                                (((1,), (1,)), ((), ())),
                                preferred_element_type=jnp.float32) + b3_ref[0:1, :]
        qs_ref[...] = q * SCALE
        numer_ref[...] = jnp.zeros_like(numer_ref)
        denom_ref[...] = jnp.zeros_like(denom_ref)
        cnt_ref[...] = jnp.zeros_like(cnt_ref)

    feat = feat_ref[...]
    k_t = jax.lax.dot_general(feat, ipw_ref[C:2 * C, :],
                              (((1,), (1,)), ((), ())),
                              preferred_element_type=jnp.float32) + b3_ref[1:2, :]
    v_t = jax.lax.dot_general(feat, ipw_ref[2 * C:3 * C, :],
                              (((1,), (1,)), ((), ())),
                              preferred_element_type=jnp.float32) + b3_ref[2:3, :]

    bidx = bidx_ref[0]
    row_b = jax.lax.broadcasted_iota(jnp.int32, (B * M, TT), 0) // M
    maskE = (row_b == bidx).astype(jnp.float32)
    oh_bt = (jax.lax.broadcasted_iota(jnp.int32, (B, TT), 0)
             == bidx).astype(jnp.float32)
    cnt_new = cnt_ref[...] + oh_bt

    qs = qs_ref[...]
    nparts = []
    dparts = []
    for h in range(H):
        k_h = k_t[:, h * DH:(h + 1) * DH]
        v_h = v_t[:, h * DH:(h + 1) * DH]
        s_h = jax.lax.dot_general(qs[:, h * DH:(h + 1) * DH], k_h,
                                  (((1,), (1,)), ((), ())),
                                  preferred_element_type=jnp.float32)
        e_h = jnp.exp(s_h)
        e_tiled = jnp.concatenate([e_h] * B, axis=0)
        E = e_tiled * maskE
        nparts.append(jnp.dot(E, v_h,
                              preferred_element_type=jnp.float32))
        dparts.append(jnp.sum(E, axis=1, keepdims=True))
    numer_ref[...] += jnp.concatenate(nparts, axis=1)
    denom_ref[...] += jnp.concatenate(dparts, axis=1)
    cnt_ref[...] = cnt_new

    @pl.when(i == NT - 1)
    def _finalize():
        expmat = (jax.lax.broadcasted_iota(jnp.int32, (H, C), 0)
                  == jax.lax.broadcasted_iota(jnp.int32, (H, C), 1) // DH
                  ).astype(jnp.float32)
        d = denom_ref[...]
        inv = 1.0 / jnp.where(d == 0.0, 1.0, d)
        invx = jnp.dot(inv, expmat,
                       preferred_element_type=jnp.float32)
        ctx = numer_ref[...] * invx
        attn = jax.lax.dot_general(ctx, owT_ref[...], (((1,), (1,)), ((), ())),
                                   preferred_element_type=jnp.float32) + ob_ref[...]
        outr = jax.lax.dot_general(attn, lwT_ref[...], (((1,), (1,)), ((), ())),
                                   preferred_element_type=jnp.float32) + lb_ref[...]
        attn_d = jax.lax.dot_general(b3_ref[2:3, :], owT_ref[...],
                                     (((1,), (1,)), ((), ())),
                                     preferred_element_type=jnp.float32) + ob_ref[...]
        out_d = jax.lax.dot_general(attn_d, lwT_ref[...],
                                    (((1,), (1,)), ((), ())),
                                    preferred_element_type=jnp.float32) + lb_ref[...]
        alive = (jnp.sum(cnt_ref[...], axis=1, keepdims=True)
                 > 0.0).astype(jnp.float32)
        rowsel = (jax.lax.broadcasted_iota(jnp.int32, (B * M, B), 0) // M
                  == jax.lax.broadcasted_iota(jnp.int32, (B * M, B), 1)
                  ).astype(jnp.float32)
        keep = jnp.dot(rowsel, alive,
                       preferred_element_type=jnp.float32)
        out_ref[...] = keep * outr + (1.0 - keep) * out_d


def _run(bidx3, feat, q2, ipw, b3, owT, ob2, lwT, lb2):
    return pl.pallas_call(
        _fused_kernel,
        grid=(NT,),
        in_specs=[
            pl.BlockSpec((1, 1, TT), lambda i: (i, 0, 0)),
            pl.BlockSpec((TT, C), lambda i: (i, 0)),
            pl.BlockSpec((M, C), lambda i: (0, 0)),
            pl.BlockSpec((3 * C, C), lambda i: (0, 0)),
            pl.BlockSpec((3, C), lambda i: (0, 0)),
            pl.BlockSpec((C, C), lambda i: (0, 0)),
            pl.BlockSpec((1, C), lambda i: (0, 0)),
            pl.BlockSpec((K, C), lambda i: (0, 0)),
            pl.BlockSpec((1, K), lambda i: (0, 0)),
        ],
        out_specs=pl.BlockSpec((B * M, K), lambda i: (0, 0)),
        out_shape=jax.ShapeDtypeStruct((B * M, K), jnp.float32),
        scratch_shapes=[
            pltpu.VMEM((M, C), jnp.float32),
            pltpu.VMEM((B * M, C), jnp.float32),
            pltpu.VMEM((B * M, H), jnp.float32),
            pltpu.VMEM((B, TT), jnp.float32),
        ],
        compiler_params=pltpu.CompilerParams(
            dimension_semantics=("arbitrary",),
        ),
    )(bidx3, feat, q2, ipw, b3, owT, ob2, lwT, lb2)


def kernel(feat_all, batch_idx, queries, in_proj_w, in_proj_b, out_w, out_b,
           lin_w, lin_b):
    bidx3 = batch_idx.astype(jnp.int32).reshape(NT, 1, TT)
    q2 = queries.reshape(M, C)
    b3 = in_proj_b.reshape(3, C)
    ob2 = out_b.reshape(1, C)
    lb2 = lin_b.reshape(1, K)
    out = _run(bidx3, feat_all, q2, in_proj_w, b3, out_w, ob2, lin_w, lb2)
    return out.reshape(B, M, K)


# speedup vs baseline: 1.4376x; 1.1447x over previous
import functools

import jax
import jax.numpy as jnp
import numpy as np
from jax.experimental import pallas as pl
from jax.experimental.pallas import tpu as pltpu

C = 1024
M = 64
K = 512
H = 8
B = 8
T = 8192
DH = C // H
DA = 2 * DH
TT = 1024
NT = T // TT
SCALE = float(1.0 / np.sqrt(DH))


def _fused_kernel(bidx_ref, feat_ref, q_ref, ipw_ref, b3_ref,
                  ow_ref, ob_ref, lw_ref, lb_ref, out_ref,
                  qs_ref, acc_ref):
    i = pl.program_id(0)

    @pl.when(i == 0)
    def _init():
        q = jax.lax.dot_general(q_ref[...], ipw_ref[0:C, :],
                                (((1,), (1,)), ((), ())),
                                preferred_element_type=jnp.float32) + b3_ref[0:1, :]
        qs_ref[...] = q * SCALE
        acc_ref[...] = jnp.zeros_like(acc_ref)

    feat = feat_ref[...]
    k_t = jax.lax.dot_general(feat, ipw_ref[C:2 * C, :],
                              (((1,), (1,)), ((), ())),
                              preferred_element_type=jnp.float32) + b3_ref[1:2, :]
    v_t = jax.lax.dot_general(feat, ipw_ref[2 * C:3 * C, :],
                              (((1,), (1,)), ((), ())),
                              preferred_element_type=jnp.float32)

    bidx = bidx_ref[0]
    row_b = jax.lax.broadcasted_iota(jnp.int32, (B * M, TT), 0) // M
    maskE = (row_b == bidx).astype(jnp.float32)
    onescol = (jax.lax.broadcasted_iota(jnp.int32, (TT, DH), 1)
               == 0).astype(jnp.float32)

    qs = qs_ref[...]
    parts = []
    for h in range(H):
        k_h = k_t[:, h * DH:(h + 1) * DH]
        v_a = jnp.concatenate([v_t[:, h * DH:(h + 1) * DH], onescol],
                              axis=1)
        s_h = jax.lax.dot_general(qs[:, h * DH:(h + 1) * DH], k_h,
                                  (((1,), (1,)), ((), ())),
                                  preferred_element_type=jnp.float32)
        e_h = jnp.exp(s_h)
        e_tiled = jnp.concatenate([e_h] * B, axis=0)
        E = e_tiled * maskE
        parts.append(jnp.dot(E, v_a,
                             preferred_element_type=jnp.float32))
    acc_ref[...] += jnp.concatenate(parts, axis=1)

    @pl.when(i == NT - 1)
    def _finalize():
        numer = jnp.concatenate(
            [acc_ref[:, h * DA:h * DA + DH] for h in range(H)],
            axis=1)
        d = jnp.concatenate(
            [acc_ref[:, h * DA + DH:h * DA + DH + 1] for h in range(H)],
            axis=1)
        inv = 1.0 / jnp.where(d == 0.0, 1.0, d)
        expmat = (jax.lax.broadcasted_iota(jnp.int32, (H, C), 0)
                  == jax.lax.broadcasted_iota(jnp.int32, (H, C), 1) // DH
                  ).astype(jnp.float32)
        invx = jnp.dot(inv, expmat,
                       preferred_element_type=jnp.float32)
        ctx = numer * invx + b3_ref[2:3, :]
        attn = jax.lax.dot_general(ctx, ow_ref[...], (((1,), (1,)), ((), ())),
                                   preferred_element_type=jnp.float32) + ob_ref[...]
        out_ref[...] = jax.lax.dot_general(
            attn, lw_ref[...], (((1,), (1,)), ((), ())),
            preferred_element_type=jnp.float32) + lb_ref[...]


def _run(bidx3, feat, q2, ipw, b3, ow, ob2, lw, lb2):
    return pl.pallas_call(
        _fused_kernel,
        grid=(NT,),
        in_specs=[
            pl.BlockSpec((1, 1, TT), lambda i: (i, 0, 0)),
            pl.BlockSpec((TT, C), lambda i: (i, 0)),
            pl.BlockSpec((M, C), lambda i: (0, 0)),
            pl.BlockSpec((3 * C, C), lambda i: (0, 0)),
            pl.BlockSpec((3, C), lambda i: (0, 0)),
            pl.BlockSpec((C, C), lambda i: (0, 0)),
            pl.BlockSpec((1, C), lambda i: (0, 0)),
            pl.BlockSpec((K, C), lambda i: (0, 0)),
            pl.BlockSpec((1, K), lambda i: (0, 0)),
        ],
        out_specs=pl.BlockSpec((B * M, K), lambda i: (0, 0)),
        out_shape=jax.ShapeDtypeStruct((B * M, K), jnp.float32),
        scratch_shapes=[
            pltpu.VMEM((M, C), jnp.float32),
            pltpu.VMEM((B * M, H * DA), jnp.float32),
        ],
        compiler_params=pltpu.CompilerParams(
            dimension_semantics=("arbitrary",),
        ),
    )(bidx3, feat, q2, ipw, b3, ow, ob2, lw, lb2)


def kernel(feat_all, batch_idx, queries, in_proj_w, in_proj_b, out_w, out_b,
           lin_w, lin_b):
    bidx3 = batch_idx.astype(jnp.int32).reshape(NT, 1, TT)
    q2 = queries.reshape(M, C)
    b3 = in_proj_b.reshape(3, C)
    ob2 = out_b.reshape(1, C)
    lb2 = lin_b.reshape(1, K)
    out = _run(bidx3, feat_all, q2, in_proj_w, b3, out_w, ob2, lin_w, lb2)
    return out.reshape(B, M, K)
